# baseline (speedup 1.0000x reference)
import functools

import jax
import jax.numpy as jnp
from jax import lax
from jax.experimental import pallas as pl
from jax.experimental.pallas import tpu as pltpu
from jax.experimental.pallas import tpu_sc as plsc

NN = 10000
NP = 10240
NPS = NP // 16
BB = 128
CC = 128
FW = 128


@functools.lru_cache(maxsize=None)
def _make_agg(CH, nrows):
    mesh = plsc.VectorSubcoreMesh(core_axis_name="c", subcore_axis_name="s")

    KB = 16
    NBLK = CH // KB

    @functools.partial(
        pl.kernel, mesh=mesh,
        out_type=jax.ShapeDtypeStruct((2 * NP, FW), jnp.float32),
        scratch_types=[
            pltpu.VMEM((2, KB, CC), jnp.int32),
            pltpu.VMEM((2, KB, CC), jnp.int32),
            pltpu.VMEM((2, CC, FW), jnp.float32),
            pltpu.VMEM_SHARED((NP, FW), jnp.float32),
            pltpu.SemaphoreType.DMA((2,)),
            pltpu.SemaphoreType.DMA((2,)),
            pltpu.SemaphoreType.DMA,
        ])
    def agg(table_h, srcp_h, dstp_h, zrows_h, out_h, src_v, dst_v, gbuf,
            accum, gsem, ssem, isem):
        c = lax.axis_index("c")
        s = lax.axis_index("s")
        w = c * 16 + s
        pltpu.sync_copy(srcp_h.at[w, pl.ds(0, KB)], src_v.at[0])
        pltpu.sync_copy(dstp_h.at[w, pl.ds(0, KB)], dst_v.at[0])
        pltpu.sync_copy(zrows_h, accum.at[pl.ds(s * NPS, NPS)])
        plsc.subcore_barrier()
        pltpu.async_copy(table_h.at[src_v.at[0, 0]], gbuf.at[0], gsem.at[0])

        def body(j, carry):
            blk = j // KB
            off = j - blk * KB
            par = blk % 2
            cur = j % 2
            nxt = 1 - cur
            jn = j + 1
            blkn = jn // KB
            offn = jn - blkn * KB
            parn = blkn % 2

            @pl.when(j >= 1)
            def _():
                pltpu.make_async_copy(
                    gbuf.at[nxt], accum.at[dst_v.at[0, 0]],
                    ssem.at[nxt]).wait()

            @pl.when(jnp.logical_and(off == 0, blk + 1 < NBLK))
            def _():
                pltpu.async_copy(srcp_h.at[w, pl.ds((blk + 1) * KB, KB)],
                                 src_v.at[1 - par], isem)
                pltpu.async_copy(dstp_h.at[w, pl.ds((blk + 1) * KB, KB)],
                                 dst_v.at[1 - par], isem)

            @pl.when(jnp.logical_and(off == KB - 1, jn < CH))
            def _():
                pltpu.make_async_copy(srcp_h.at[w, pl.ds(0, KB)],
                                      src_v.at[0], isem).wait()
                pltpu.make_async_copy(dstp_h.at[w, pl.ds(0, KB)],
                                      dst_v.at[0], isem).wait()

            @pl.when(jn < CH)
            def _():
                pltpu.async_copy(table_h.at[src_v.at[parn, offn]],
                                 gbuf.at[nxt], gsem.at[nxt])

            pltpu.make_async_copy(table_h.at[src_v.at[par, off]],
                                  gbuf.at[cur], gsem.at[cur]).wait()
            pltpu.async_copy(gbuf.at[cur], accum.at[dst_v.at[par, off]],
                             ssem.at[cur], add=True)
            return carry

        lax.fori_loop(0, CH, body, 0)
        pltpu.make_async_copy(gbuf.at[(CH - 1) % 2],
                              accum.at[dst_v.at[0, 0]],
                              ssem.at[(CH - 1) % 2]).wait()
        plsc.subcore_barrier()
        pltpu.sync_copy(accum.at[pl.ds(s * NPS, NPS)],
                        out_h.at[pl.ds(c * NP + s * NPS, NPS)])

    return agg


@functools.lru_cache(maxsize=None)
def _make_deg(CH):
    mesh = plsc.VectorSubcoreMesh(core_axis_name="c", subcore_axis_name="s")

    KB = 16
    NBLK = CH // KB

    DEPTH = 2 * KB

    @functools.partial(
        pl.kernel, mesh=mesh,
        out_type=jax.ShapeDtypeStruct((2 * NP, FW), jnp.float32),
        scratch_types=[
            pltpu.VMEM((4, KB, CC), jnp.int32),
            pltpu.VMEM((CC, FW), jnp.float32),
            pltpu.VMEM_SHARED((NP, FW), jnp.float32),
            pltpu.SemaphoreType.DMA,
            pltpu.SemaphoreType.DMA,
        ])
    def deg(dstp_h, ones_h, zrows_h, out_h, dst_v, ones_v, accum, ssem,
            isem):
        c = lax.axis_index("c")
        s = lax.axis_index("s")
        w = c * 16 + s
        pltpu.sync_copy(ones_h, ones_v)
        pltpu.sync_copy(dstp_h.at[w, pl.ds(0, KB)], dst_v.at[0])
        pltpu.sync_copy(zrows_h, accum.at[pl.ds(s * NPS, NPS)])
        plsc.subcore_barrier()

        def body(j, carry):
            blk = j // KB
            off = j - blk * KB
            par = blk % 4

            @pl.when(jnp.logical_and(off == 0, blk + 1 < NBLK))
            def _():
                pltpu.async_copy(dstp_h.at[w, pl.ds((blk + 1) * KB, KB)],
                                 dst_v.at[(blk + 1) % 4], isem)

            @pl.when(jnp.logical_and(off == KB - 1, j + 1 < CH))
            def _():
                pltpu.make_async_copy(dstp_h.at[w, pl.ds(0, KB)],
                                      dst_v.at[0], isem).wait()

            @pl.when(j >= DEPTH)
            def _():
                pltpu.make_async_copy(ones_v, accum.at[dst_v.at[0, 0]],
                                      ssem).wait()

            pltpu.async_copy(ones_v, accum.at[dst_v.at[par, off]], ssem,
                             add=True)
            return carry

        lax.fori_loop(0, CH, body, 0)

        def drain(j, carry):
            pltpu.make_async_copy(ones_v, accum.at[dst_v.at[0, 0]],
                                  ssem).wait()
            return carry

        lax.fori_loop(0, min(DEPTH, CH), drain, 0)
        plsc.subcore_barrier()
        pltpu.sync_copy(accum.at[pl.ds(s * NPS, NPS)],
                        out_h.at[pl.ds(c * NP + s * NPS, NPS)])

    return deg


def _prep_call(x_pad, deg2):
    BN = 512

    def body(x_ref, d_ref, dinv_ref, ts_ref):
        i = pl.program_id(0)
        deg = d_ref[0][:, 0:1] + d_ref[1][:, 0:1] + 1.0
        row = i * BN + lax.broadcasted_iota(jnp.int32, (BN, 1), 0)
        dinv = jnp.where(row < NN, lax.rsqrt(deg), 0.0)
        dinv_ref[...] = dinv
        ts_ref[...] = x_ref[...] * dinv

    return pl.pallas_call(
        body, grid=(NP // BN,),
        in_specs=[pl.BlockSpec((BN, 128), lambda i: (i, 0)),
                  pl.BlockSpec((2, BN, FW), lambda i: (0, i, 0))],
        out_specs=[pl.BlockSpec((BN, 1), lambda i: (i, 0)),
                   pl.BlockSpec((BN, 128), lambda i: (i, 0))],
        out_shape=[jax.ShapeDtypeStruct((NP, 1), jnp.float32),
                   jax.ShapeDtypeStruct((NP, 128), jnp.float32)],
    )(x_pad, deg2)


def _mm_p_call(P, ts, dinv, W, bias, split_out):
    BN = 512
    Fi, Fo = W.shape

    def body(P_ref, ts_ref, dinv_ref, W_ref, b_ref, out_ref):
        dinv = dinv_ref[...]
        a = (P_ref[0] + P_ref[1] + ts_ref[...]) * dinv
        h = jnp.maximum(
            jnp.dot(a, W_ref[...], preferred_element_type=jnp.float32)
            + b_ref[...], 0.0)
        hs = h * dinv
        if split_out:
            out_ref[0] = hs[:, :Fo // 2]
            out_ref[1] = hs[:, Fo // 2:]
        else:
            out_ref[...] = hs

    if split_out:
        out_specs = pl.BlockSpec((2, BN, Fo // 2), lambda i: (0, i, 0))
        out_shape = jax.ShapeDtypeStruct((2, NP, Fo // 2), jnp.float32)
    else:
        out_specs = pl.BlockSpec((BN, Fo), lambda i: (i, 0))
        out_shape = jax.ShapeDtypeStruct((NP, Fo), jnp.float32)
    return pl.pallas_call(
        body, grid=(NP // BN,),
        in_specs=[pl.BlockSpec((2, BN, Fi), lambda i: (0, i, 0)),
                  pl.BlockSpec((BN, Fi), lambda i: (i, 0)),
                  pl.BlockSpec((BN, 1), lambda i: (i, 0)),
                  pl.BlockSpec((Fi, Fo), lambda i: (0, 0)),
                  pl.BlockSpec((1, Fo), lambda i: (0, 0))],
        out_specs=out_specs, out_shape=out_shape,
    )(P, ts, dinv, W, bias)


def _mm_s_call(S, ts, dinv, W, bias):
    BN = 512
    Fi, Fo = W.shape

    def body(S_ref, ts_ref, dinv_ref, W_ref, b_ref, out_ref):
        a = jnp.concatenate([S_ref[0] + ts_ref[0], S_ref[1] + ts_ref[1]],
                            axis=1) * dinv_ref[...]
        out_ref[...] = jnp.maximum(
            jnp.dot(a, W_ref[...], preferred_element_type=jnp.float32)
            + b_ref[...], 0.0)

    return pl.pallas_call(
        body, grid=(NP // BN,),
        in_specs=[pl.BlockSpec((2, BN, Fi // 2), lambda i: (0, i, 0)),
                  pl.BlockSpec((2, BN, Fi // 2), lambda i: (0, i, 0)),
                  pl.BlockSpec((BN, 1), lambda i: (i, 0)),
                  pl.BlockSpec((Fi, Fo), lambda i: (0, 0)),
                  pl.BlockSpec((1, Fo), lambda i: (0, 0))],
        out_specs=pl.BlockSpec((BN, Fo), lambda i: (i, 0)),
        out_shape=jax.ShapeDtypeStruct((NP, Fo), jnp.float32),
    )(S, ts, dinv, W, bias)


def _pool_call(batch2d, h3):
    FF = h3.shape[1]

    def body(b_ref, h_ref, out_ref):
        b = pl.program_id(0)
        bt = b_ref[...]
        start = jnp.sum((bt < b).astype(jnp.int32))
        end = jnp.sum((bt <= b).astype(jnp.int32))

        def it(k, acc):
            rows = h_ref[pl.ds(k * 8, 8), :]
            rid = k * 8 + lax.broadcasted_iota(jnp.int32, (8, 1), 0)
            m = (rid >= start) & (rid < end)
            return jnp.maximum(
                acc, jnp.max(jnp.where(m, rows, -jnp.inf), axis=0,
                             keepdims=True))

        acc = jnp.full((1, FF), -jnp.inf, jnp.float32)
        out_ref[0] = lax.fori_loop(start // 8, (end + 7) // 8, it, acc)

    return pl.pallas_call(
        body, grid=(BB,),
        in_specs=[pl.BlockSpec((1, NN), lambda b: (0, 0)),
                  pl.BlockSpec(h3.shape, lambda b: (0, 0))],
        out_specs=pl.BlockSpec((1, 1, FF), lambda b: (b, 0, 0)),
        out_shape=jax.ShapeDtypeStruct((BB, 1, FF), jnp.float32),
    )(batch2d, h3).reshape(BB, FF)


def _head_kernel(g1_ref, g2_ref, tgt_ref,
                 d1fw1_ref, d1fb1_ref, d1fw2_ref, d1fb2_ref,
                 d2fw1_ref, d2fb1_ref, d2fw2_ref, d2fb2_ref,
                 xtw_ref, xtb_ref, f1w_ref, f1b_ref,
                 f2w_ref, f2b_ref, ow_ref, ob_ref, out_ref):
    g1 = jax.nn.relu(jnp.dot(g1_ref[...], d1fw1_ref[...],
                             preferred_element_type=jnp.float32)
                     + d1fb1_ref[...])
    g1 = jnp.dot(g1, d1fw2_ref[...],
                 preferred_element_type=jnp.float32) + d1fb2_ref[...]
    g2 = jax.nn.relu(jnp.dot(g2_ref[...], d2fw1_ref[...],
                             preferred_element_type=jnp.float32)
                     + d2fb1_ref[...])
    g2 = jnp.dot(g2, d2fw2_ref[...],
                 preferred_element_type=jnp.float32) + d2fb2_ref[...]
    xt = jnp.dot(tgt_ref[...], xtw_ref[...],
                 preferred_element_type=jnp.float32) + xtb_ref[...]
    xc = jnp.concatenate([g1, g2, xt], axis=1)
    xc = jax.nn.relu(jnp.dot(xc, f1w_ref[...],
                             preferred_element_type=jnp.float32) + f1b_ref[...])
    xc = jax.nn.relu(jnp.dot(xc, f2w_ref[...],
                             preferred_element_type=jnp.float32) + f2b_ref[...])
    out_ref[...] = jnp.dot(xc, ow_ref[...],
                           preferred_element_type=jnp.float32) + ob_ref[...]


def _junk_rows(n):
    return NN + (jnp.arange(n, dtype=jnp.int32) % (NP - NN))


def _pack(ei):
    E = ei.shape[1]
    src, dst = ei[0], ei[1]
    CHD = 16 * -(-E // (32 * CC * 16))
    EDp = 32 * CHD * CC
    srcd = jnp.concatenate([src, _junk_rows(EDp - E)]).reshape(32, CHD, CC)
    dstd = jnp.concatenate([dst, _junk_rows(EDp - E)]).reshape(32, CHD, CC)

    CHA = 16 * -(-E // (16 * CC * 16))
    EAp = 16 * CHA * CC
    src0 = jnp.concatenate([src, _junk_rows(EAp - E)]).reshape(16, CHA, CC)
    srcf = jnp.concatenate([src0, src0 + NP], axis=0)
    dst0 = jnp.concatenate([dst, _junk_rows(EAp - E)]).reshape(16, CHA, CC)
    dstf = jnp.concatenate([dst0, dst0], axis=0)
    return CHD, srcd, dstd, CHA, srcf, dstf


def kernel(x1, edge_index1, batch1, x2, edge_index2, batch2, target,
           d1w1, d1b1, d1w2, d1b2, d1w3, d1b3, d1fw1, d1fb1, d1fw2, d1fb2,
           d2w1, d2b1, d2w2, d2b2, d2w3, d2b3, d2fw1, d2fb1, d2fw2, d2fb2,
           xtw, xtb, f1w, f1b, f2w, f2b, ow, ob):
    ones = jnp.ones((CC, FW), jnp.float32)
    zrows = jnp.zeros((NPS, FW), jnp.float32)
    packs = [_pack(edge_index1), _pack(edge_index2)]
    xs = [x1, x2]
    bts = [batch1, batch2]
    wss = [(d1w1, d1b1, d1w2, d1b2, d1w3, d1b3),
           (d2w1, d2b1, d2w2, d2b2, d2w3, d2b3)]

    degs = [_make_deg(p[0])(p[2], ones, zrows).reshape(2, NP, FW)
            for p in packs]
    preps = [_prep_call(jnp.pad(xs[i], ((0, NP - NN), (0, 0))), degs[i])
             for i in range(2)]

    Ps = [_make_agg(packs[i][0], NP)(preps[i][1], packs[i][1], packs[i][2],
                                     zrows).reshape(2, NP, FW)
          for i in range(2)]
    ts1s = [_mm_p_call(Ps[i], preps[i][1], preps[i][0], wss[i][0],
                       wss[i][1].reshape(1, -1), split_out=False)
            for i in range(2)]
    P2s = [_make_agg(packs[i][0], NP)(ts1s[i], packs[i][1], packs[i][2],
                                      zrows).reshape(2, NP, FW)
           for i in range(2)]
    ts2s = [_mm_p_call(P2s[i], ts1s[i], preps[i][0], wss[i][2],
                       wss[i][3].reshape(1, -1), split_out=True)
            for i in range(2)]
    S3s = [_make_agg(packs[i][3], 2 * NP)(ts2s[i].reshape(2 * NP, FW),
                                              packs[i][4], packs[i][5],
                                              zrows).reshape(2, NP, FW)
           for i in range(2)]
    h3s = [_mm_s_call(S3s[i], ts2s[i], preps[i][0], wss[i][4],
                      wss[i][5].reshape(1, -1))
           for i in range(2)]
    g1, g2 = [_pool_call(bts[i].reshape(1, NN), h3s[i]) for i in range(2)]
    out = pl.pallas_call(
        _head_kernel,
        out_shape=jax.ShapeDtypeStruct((BB, 1), jnp.float32),
    )(g1, g2, target,
      d1fw1, d1fb1, d1fw2, d1fb2,
      d2fw1, d2fb1, d2fw2, d2fb2,
      xtw, xtb, f1w, f1b, f2w, f2b, ow, ob)
    return out
